# baseline (speedup 1.0000x reference)
import jax
import jax.numpy as jnp
from jax.experimental import pallas as pl

_ALPHA = 0.3


def _draw_kernel(cid_ref, colorsT_ref, masks_ref, img_ref, out_ref):
    num_classes = colorsT_ref.shape[1]
    cid = cid_ref[0]
    iota = jax.lax.broadcasted_iota(jnp.int32, (num_classes, cid.shape[1]), 0)
    onehot = (iota == cid).astype(jnp.float32)
    m = masks_ref[0]
    seg = jax.lax.dot_general(
        onehot, m, (((1,), (0,)), ((), ())),
        preferred_element_type=jnp.float32,
        precision=jax.lax.Precision.HIGHEST,
    )
    segb = (seg > 0.5).astype(jnp.float32)
    color = jax.lax.dot_general(
        colorsT_ref[...], segb, (((1,), (0,)), ((), ())),
        preferred_element_type=jnp.float32,
    )
    vis = img_ref[0] + color * jnp.float32(_ALPHA)
    out_ref[0] = jnp.clip(vis, 0.0, 255.0).astype(jnp.uint8)


def kernel(images, det_outs, crop_and_padded_masks, colors):
    B, H, W, _ = images.shape
    N = crop_and_padded_masks.shape[1]
    C = colors.shape[0]
    HW = H * W

    cid = det_outs[..., -2].astype(jnp.int32).reshape(B, 1, N)
    colorsT = colors.T
    masks = crop_and_padded_masks.reshape(B, N, HW)
    imgp = jnp.transpose(images, (0, 3, 1, 2)).reshape(B, 3, HW)

    n_tiles = 12
    T = HW // n_tiles

    out = pl.pallas_call(
        _draw_kernel,
        grid=(B, n_tiles),
        in_specs=[
            pl.BlockSpec((1, 1, N), lambda b, t: (b, 0, 0)),
            pl.BlockSpec((3, C), lambda b, t: (0, 0)),
            pl.BlockSpec((1, N, T), lambda b, t: (b, 0, t)),
            pl.BlockSpec((1, 3, T), lambda b, t: (b, 0, t)),
        ],
        out_specs=pl.BlockSpec((1, 3, T), lambda b, t: (b, 0, t)),
        out_shape=jax.ShapeDtypeStruct((B, 3, HW), jnp.uint8),
    )(cid, colorsT, masks, imgp)

    return jnp.transpose(out.reshape(B, 3, H, W), (0, 2, 3, 1))


# speedup vs baseline: 22.2363x; 22.2363x over previous
import jax
import jax.numpy as jnp
from jax.experimental import pallas as pl
from jax.experimental.pallas import tpu as pltpu

_ALPHA = 0.3


def _make_kernel(N, C, TH, W):
    def _draw_kernel(cid_ref, colors_ref, masks_ref, img_ref, out_ref, acc_ref):
        b = pl.program_id(0)
        acc_ref[...] = jnp.zeros_like(acc_ref)
        for n in range(N):
            c = cid_ref[b, n]
            acc_ref[pl.ds(c, 1), :, :] = (
                acc_ref[pl.ds(c, 1), :, :] + masks_ref[0, n][None, :, :]
            )
        seg = acc_ref[...]
        segb = seg > 0.5
        for k in range(3):
            tot = jnp.zeros((TH, W), jnp.float32)
            for c in range(C):
                tot = tot + jnp.where(segb[c], colors_ref[c, k], jnp.float32(0.0))
            vis = img_ref[0, k] + tot * jnp.float32(_ALPHA)
            out_ref[0, k] = jnp.clip(vis, 0.0, 255.0).astype(jnp.uint8)

    return _draw_kernel


def kernel(images, det_outs, crop_and_padded_masks, colors):
    B, H, W, _ = images.shape
    N = crop_and_padded_masks.shape[1]
    C = colors.shape[0]

    cid = det_outs[..., -2].astype(jnp.int32)
    imgp = jnp.transpose(images, (0, 3, 1, 2))

    TH = 32
    out = pl.pallas_call(
        _make_kernel(N, C, TH, W),
        grid=(B, H // TH),
        in_specs=[
            pl.BlockSpec(memory_space=pltpu.SMEM),
            pl.BlockSpec(memory_space=pltpu.SMEM),
            pl.BlockSpec((1, N, TH, W), lambda b, t: (b, 0, t, 0)),
            pl.BlockSpec((1, 3, TH, W), lambda b, t: (b, 0, t, 0)),
        ],
        out_specs=pl.BlockSpec((1, 3, TH, W), lambda b, t: (b, 0, t, 0)),
        out_shape=jax.ShapeDtypeStruct((B, 3, H, W), jnp.uint8),
        scratch_shapes=[pltpu.VMEM((C, TH, W), jnp.float32)],
    )(cid, colors, crop_and_padded_masks, imgp)

    return jnp.transpose(out, (0, 2, 3, 1))


# speedup vs baseline: 27.0889x; 1.2182x over previous
import jax
import jax.numpy as jnp
from jax.experimental import pallas as pl
from jax.experimental.pallas import tpu as pltpu

_ALPHA = 0.3


def _make_kernel(N, C, TH, W):
    def _draw_kernel(cid_ref, colors_ref, masks_ref, img_ref, out_ref, acc_ref):
        b = pl.program_id(0)
        acc_ref[...] = jnp.zeros_like(acc_ref)
        for n in range(N):
            c = cid_ref[b, n]
            acc_ref[pl.ds(c, 1), :, :] = (
                acc_ref[pl.ds(c, 1), :, :] + masks_ref[0, n][None, :, :]
            )
        seg = acc_ref[...]
        segb = seg > 0.5
        for k in range(3):
            tot = jnp.zeros((TH, W), jnp.float32)
            for c in range(C):
                tot = tot + jnp.where(segb[c], colors_ref[c, k], jnp.float32(0.0))
            vis = img_ref[0, k] + tot * jnp.float32(_ALPHA)
            out_ref[0, k] = jnp.clip(vis, 0.0, 255.0).astype(jnp.uint8)

    return _draw_kernel


def kernel(images, det_outs, crop_and_padded_masks, colors):
    B, H, W, _ = images.shape
    N = crop_and_padded_masks.shape[1]
    C = colors.shape[0]

    cid = det_outs[..., -2].astype(jnp.int32)
    imgp = jnp.transpose(images, (0, 3, 1, 2))

    TH = 64
    out = pl.pallas_call(
        _make_kernel(N, C, TH, W),
        grid=(B, H // TH),
        in_specs=[
            pl.BlockSpec(memory_space=pltpu.SMEM),
            pl.BlockSpec(memory_space=pltpu.SMEM),
            pl.BlockSpec((1, N, TH, W), lambda b, t: (b, 0, t, 0)),
            pl.BlockSpec((1, 3, TH, W), lambda b, t: (b, 0, t, 0)),
        ],
        out_specs=pl.BlockSpec((1, 3, TH, W), lambda b, t: (b, 0, t, 0)),
        out_shape=jax.ShapeDtypeStruct((B, 3, H, W), jnp.uint8),
        scratch_shapes=[pltpu.VMEM((C, TH, W), jnp.float32)],
    )(cid, colors, crop_and_padded_masks, imgp)

    return jnp.transpose(out, (0, 2, 3, 1))


# speedup vs baseline: 27.6693x; 1.0214x over previous
import jax
import jax.numpy as jnp
from jax.experimental import pallas as pl
from jax.experimental.pallas import tpu as pltpu

_ALPHA = 0.3


def _make_kernel(N, C, TH, W):
    def _draw_kernel(cid_ref, colors_ref, masks_ref, img_ref, out_ref, acc_ref):
        b = pl.program_id(0)
        acc_ref[...] = jnp.zeros_like(acc_ref)
        for n in range(N):
            c = cid_ref[b, n]
            acc_ref[pl.ds(c, 1), :, :] = (
                acc_ref[pl.ds(c, 1), :, :] + masks_ref[0, n][None, :, :]
            )
        seg = acc_ref[...]
        segb = seg > 0.5
        for k in range(3):
            tot = jnp.zeros((TH, W), jnp.float32)
            for c in range(C):
                tot = tot + jnp.where(segb[c], colors_ref[c, k], jnp.float32(0.0))
            vis = img_ref[0, k] + tot * jnp.float32(_ALPHA)
            out_ref[0, k] = jnp.clip(vis, 0.0, 255.0).astype(jnp.uint8)

    return _draw_kernel


def kernel(images, det_outs, crop_and_padded_masks, colors):
    B, H, W, _ = images.shape
    N = crop_and_padded_masks.shape[1]
    C = colors.shape[0]

    cid = det_outs[..., -2].astype(jnp.int32)
    imgp = jnp.transpose(images, (0, 3, 1, 2))

    TH = 128
    out = pl.pallas_call(
        _make_kernel(N, C, TH, W),
        grid=(B, H // TH),
        in_specs=[
            pl.BlockSpec(memory_space=pltpu.SMEM),
            pl.BlockSpec(memory_space=pltpu.SMEM),
            pl.BlockSpec((1, N, TH, W), lambda b, t: (b, 0, t, 0)),
            pl.BlockSpec((1, 3, TH, W), lambda b, t: (b, 0, t, 0)),
        ],
        out_specs=pl.BlockSpec((1, 3, TH, W), lambda b, t: (b, 0, t, 0)),
        out_shape=jax.ShapeDtypeStruct((B, 3, H, W), jnp.uint8),
        scratch_shapes=[pltpu.VMEM((C, TH, W), jnp.float32)],
    )(cid, colors, crop_and_padded_masks, imgp)

    return jnp.transpose(out, (0, 2, 3, 1))


# speedup vs baseline: 27.9870x; 1.0115x over previous
import jax
import jax.numpy as jnp
from jax.experimental import pallas as pl
from jax.experimental.pallas import tpu as pltpu

_ALPHA = 0.3


def _make_kernel(N, C, TH, W):
    def _draw_kernel(cid_ref, colors_ref, masks_ref, img_ref, out_ref, acc_ref):
        b = pl.program_id(0)
        acc_ref[...] = jnp.zeros_like(acc_ref)
        for n in range(N):
            c = cid_ref[b, n]
            acc_ref[pl.ds(c, 1), :, :] = (
                acc_ref[pl.ds(c, 1), :, :] + masks_ref[0, n][None, :, :]
            )
        seg = acc_ref[...]
        segb = seg > 0.5
        for k in range(3):
            tot = jnp.zeros((TH, W), jnp.float32)
            for c in range(C):
                tot = tot + jnp.where(segb[c], colors_ref[c, k], jnp.float32(0.0))
            vis = img_ref[0, k] + tot * jnp.float32(_ALPHA)
            out_ref[0, k] = jnp.clip(vis, 0.0, 255.0).astype(jnp.uint8)

    return _draw_kernel


def kernel(images, det_outs, crop_and_padded_masks, colors):
    B, H, W, _ = images.shape
    N = crop_and_padded_masks.shape[1]
    C = colors.shape[0]

    cid = det_outs[..., -2].astype(jnp.int32)
    imgp = jnp.transpose(images, (0, 3, 1, 2))

    TH = 96
    out = pl.pallas_call(
        _make_kernel(N, C, TH, W),
        grid=(B, H // TH),
        in_specs=[
            pl.BlockSpec(memory_space=pltpu.SMEM),
            pl.BlockSpec(memory_space=pltpu.SMEM),
            pl.BlockSpec((1, N, TH, W), lambda b, t: (b, 0, t, 0)),
            pl.BlockSpec((1, 3, TH, W), lambda b, t: (b, 0, t, 0)),
        ],
        out_specs=pl.BlockSpec((1, 3, TH, W), lambda b, t: (b, 0, t, 0)),
        out_shape=jax.ShapeDtypeStruct((B, 3, H, W), jnp.uint8),
        scratch_shapes=[pltpu.VMEM((C, TH, W), jnp.float32)],
    )(cid, colors, crop_and_padded_masks, imgp)

    return jnp.transpose(out, (0, 2, 3, 1))
